# baseline (speedup 1.0000x reference)
import jax
import jax.numpy as jnp
from jax import lax
from jax.experimental import pallas as pl
from jax.experimental.pallas import tpu as pltpu
from jax.experimental.pallas import tpu_sc as plsc

N_ROWS = 16384
N_COLS = 50
D = 32
B_TOTAL = N_ROWS * N_COLS
SU = 512
N_SU = B_TOTAL // SU
SU_PER_S = N_ROWS // SU

_info = plsc.get_sparse_core_info()
NC = _info.num_cores
NS = _info.num_subcores
NW = NC * NS
SU_PER_W = N_SU // NW
NBUF = 2


def _transpose_block(rows_ref, tile_ref):
    row_iota = lax.iota(jnp.int32, 16)
    for c in range(D):
        col = jnp.full((16,), c, jnp.int32)
        for k in range(SU // 16):
            vec = plsc.load_gather(rows_ref, [row_iota + 16 * k, col])
            tile_ref[c, pl.ds(16 * k, 16)] = vec


def _gather_kernel(idx_hbm, table_hbm, out_hbm, idx_v, rows_v, tile_v,
                   sem_g0, sem_g1, sem_o0, sem_o1):
    wid = lax.axis_index("s") * NC + lax.axis_index("c")
    g0 = wid * SU_PER_W
    sem_g = (sem_g0, sem_g1)
    sem_o = (sem_o0, sem_o1)

    pltpu.sync_copy(idx_hbm.at[pl.ds(SU * g0, SU * SU_PER_W)], idx_v)

    def gather_su(local_g, b):
        return pltpu.async_copy(
            table_hbm.at[idx_v.at[pl.ds(SU * local_g, SU)]],
            rows_v.at[b], sem_g[b])

    for b in range(NBUF):
        gather_su(b, b)

    def body(g, carry):
        for b in range(NBUF):
            local_g = NBUF * g + b
            gu = g0 + local_g
            pltpu.make_async_copy(
                table_hbm.at[pl.ds(0, SU), :], rows_v.at[b], sem_g[b]
            ).wait()
            @pl.when(g > 0)
            def _():
                pltpu.make_async_copy(
                    tile_v.at[b], out_hbm.at[0, :, pl.ds(0, SU)], sem_o[b]
                ).wait()
            _transpose_block(rows_v.at[b], tile_v.at[b])
            @pl.when(local_g + NBUF < SU_PER_W)
            def _():
                gather_su(local_g + NBUF, b)
            s = gu // SU_PER_S
            tj = gu - s * SU_PER_S
            pltpu.async_copy(
                tile_v.at[b], out_hbm.at[s, :, pl.ds(SU * tj, SU)],
                sem_o[b])
        return carry

    lax.fori_loop(0, SU_PER_W // NBUF, body, 0)

    for b in range(NBUF):
        pltpu.make_async_copy(
            tile_v.at[b], out_hbm.at[0, :, pl.ds(0, SU)], sem_o[b]
        ).wait()


@jax.jit
def _gather(idx, weight):
    mesh = plsc.VectorSubcoreMesh(core_axis_name="c", subcore_axis_name="s")
    return pl.kernel(
        _gather_kernel,
        out_type=jax.ShapeDtypeStruct((N_COLS, D, N_ROWS), jnp.float32),
        mesh=mesh,
        scratch_types=[
            pltpu.VMEM((SU * SU_PER_W,), jnp.int32),
            pltpu.VMEM((NBUF, SU, D), jnp.float32),
            pltpu.VMEM((NBUF, D, SU), jnp.float32),
            pltpu.SemaphoreType.DMA,
            pltpu.SemaphoreType.DMA,
            pltpu.SemaphoreType.DMA,
            pltpu.SemaphoreType.DMA,
        ],
        compiler_params=pltpu.CompilerParams(
            use_tc_tiling_on_sc=False, needs_layout_passes=False),
    )(idx, weight)


def kernel(states, weight):
    idx_sm = jnp.transpose(states).reshape(-1).astype(jnp.int32)
    t4 = _gather(idx_sm, weight)
    return jnp.transpose(t4, (2, 0, 1))


# speedup vs baseline: 1.3929x; 1.3929x over previous
import jax
import jax.numpy as jnp
from jax import lax
from jax.experimental import pallas as pl
from jax.experimental.pallas import tpu as pltpu
from jax.experimental.pallas import tpu_sc as plsc

N_ROWS = 16384
N_COLS = 50
D = 32
B_TOTAL = N_ROWS * N_COLS
SU = 256
N_SU = B_TOTAL // SU

_info = plsc.get_sparse_core_info()
NC = _info.num_cores
NS = _info.num_subcores
NW = NC * NS
SU_PER_W = N_SU // NW
NBUF = 4
DIST = 2


def _gather_kernel(idx_hbm, table_hbm, out_hbm, idx_v, rows_v,
                   sem_g0, sem_g1, sem_g2, sem_g3,
                   sem_o0, sem_o1, sem_o2, sem_o3):
    wid = lax.axis_index("s") * NC + lax.axis_index("c")
    g0 = wid * SU_PER_W
    sem_g = (sem_g0, sem_g1, sem_g2, sem_g3)
    sem_o = (sem_o0, sem_o1, sem_o2, sem_o3)

    pltpu.sync_copy(idx_hbm.at[pl.ds(SU * g0, SU * SU_PER_W)], idx_v)

    def gather_su(local_n, b):
        pltpu.async_copy(
            table_hbm.at[idx_v.at[pl.ds(SU * local_n, SU)]],
            rows_v.at[b], sem_g[b])

    def wait_gather(b):
        pltpu.make_async_copy(
            table_hbm.at[pl.ds(0, SU), :], rows_v.at[b], sem_g[b]).wait()

    def wait_write(b):
        pltpu.make_async_copy(
            rows_v.at[b], out_hbm.at[pl.ds(0, SU)], sem_o[b]).wait()

    for n in range(DIST):
        gather_su(n, n)

    def body(step, carry):
        for sub in range(NBUF):
            n = NBUF * step + sub
            b = sub
            bf = (sub + DIST) % NBUF
            wait_gather(b)
            pltpu.async_copy(
                rows_v.at[b], out_hbm.at[pl.ds(SU * (g0 + n), SU)], sem_o[b])

            @pl.when(n + DIST < SU_PER_W)
            def _():
                @pl.when(n + DIST >= NBUF)
                def _():
                    wait_write(bf)
                gather_su(n + DIST, bf)
        return carry

    lax.fori_loop(0, SU_PER_W // NBUF, body, 0)

    for n in range(SU_PER_W - NBUF, SU_PER_W):
        wait_write(n % NBUF)


@jax.jit
def _gather(idx, weight):
    mesh = plsc.VectorSubcoreMesh(core_axis_name="c", subcore_axis_name="s")
    return pl.kernel(
        _gather_kernel,
        out_type=jax.ShapeDtypeStruct((B_TOTAL, D), jnp.float32),
        mesh=mesh,
        scratch_types=[
            pltpu.VMEM((SU * SU_PER_W,), jnp.int32),
            pltpu.VMEM((NBUF, SU, D), jnp.float32),
            pltpu.SemaphoreType.DMA,
            pltpu.SemaphoreType.DMA,
            pltpu.SemaphoreType.DMA,
            pltpu.SemaphoreType.DMA,
            pltpu.SemaphoreType.DMA,
            pltpu.SemaphoreType.DMA,
            pltpu.SemaphoreType.DMA,
            pltpu.SemaphoreType.DMA,
        ],
        compiler_params=pltpu.CompilerParams(
            use_tc_tiling_on_sc=False, needs_layout_passes=False),
    )(idx, weight)


def kernel(states, weight):
    idx_sm = jnp.transpose(states).reshape(-1).astype(jnp.int32)
    t5 = _gather(idx_sm, weight).reshape(N_COLS, N_ROWS, D)
    return jnp.transpose(t5, (1, 0, 2))
